# Initial kernel scaffold; baseline (speedup 1.0000x reference)
import functools

import jax
import jax.numpy as jnp
from jax import lax
from jax.experimental import pallas as pl
from jax.experimental.pallas import tpu as pltpu
from jax.experimental.pallas import tpu_sc as plsc

NUM_CORES = 2
NUM_SUBCORES = 16
NW = NUM_CORES * NUM_SUBCORES
LANES = 16


@functools.cache
def _build(B, S, H, V):
    N = B * S
    assert N % (NW * S) == 0, "each worker must own whole sequences"
    per_w = N // NW
    n_chunk = per_w // S
    rows = S

    mesh = plsc.VectorSubcoreMesh(
        core_axis_name="c", subcore_axis_name="s",
        num_cores=NUM_CORES, num_subcores=NUM_SUBCORES)

    def body(x_hbm, emb_hbm, pos_hbm, out_hbm, idx_v, pos_v, buf_v, gsem):
        wid = lax.axis_index("s") * NUM_CORES + lax.axis_index("c")
        base = wid * per_w
        pltpu.sync_copy(pos_hbm, pos_v)

        def chunk_body(k, _):
            row0 = base + k * rows
            pltpu.sync_copy(x_hbm.at[pl.ds(row0, rows)], idx_v)
            pltpu.async_copy(emb_hbm.at[idx_v], buf_v, gsem).wait()

            def add_row(r, _):
                for c in range(H // LANES):
                    sl = pl.ds(c * LANES, LANES)
                    buf_v[r, sl] = buf_v[r, sl] + pos_v[r, sl]
                return 0

            lax.fori_loop(0, S, add_row, 0)
            pltpu.sync_copy(buf_v, out_hbm.at[pl.ds(row0, rows)])
            return 0

        lax.fori_loop(0, n_chunk, chunk_body, 0)

    return pl.kernel(
        body,
        out_type=jax.ShapeDtypeStruct((N, H), jnp.float32),
        mesh=mesh,
        scratch_types=[
            pltpu.VMEM((rows,), jnp.int32),
            pltpu.VMEM((S, H), jnp.float32),
            pltpu.VMEM((rows, H), jnp.float32),
            pltpu.SemaphoreType.DMA,
        ],
    )


def kernel(x, emb_table, pos_table):
    B, S = x.shape
    V, H = emb_table.shape
    x_flat = x.reshape(B * S).astype(jnp.int32)
    out = _build(B, S, H, V)(x_flat, emb_table, pos_table[:S])
    return out.reshape(B, S, H)


# speedup vs baseline: 3.1084x; 3.1084x over previous
import functools

import jax
import jax.numpy as jnp
from jax import lax
from jax.experimental import pallas as pl
from jax.experimental.pallas import tpu as pltpu
from jax.experimental.pallas import tpu_sc as plsc

NUM_CORES = 2
NUM_SUBCORES = 16
NW = NUM_CORES * NUM_SUBCORES
LANES = 16


@functools.cache
def _build(B, S, H, V):
    N = B * S
    assert N % (NW * S) == 0, "each worker must own whole sequences"
    per_w = N // NW
    n_chunk = per_w // S
    rows = S

    mesh = plsc.VectorSubcoreMesh(
        core_axis_name="c", subcore_axis_name="s",
        num_cores=NUM_CORES, num_subcores=NUM_SUBCORES)

    def body(x_hbm, emb_hbm, pos_hbm, out_hbm, idx_v, pos_v, buf_v, gsem):
        wid = lax.axis_index("s") * NUM_CORES + lax.axis_index("c")
        base = wid * per_w
        pltpu.sync_copy(pos_hbm, pos_v)

        def chunk_body(k, _):
            row0 = base + k * rows
            pltpu.sync_copy(x_hbm.at[pl.ds(row0, rows)], idx_v)
            pltpu.async_copy(emb_hbm.at[idx_v], buf_v, gsem).wait()

            def add_row(r, _):
                for c in range(H // LANES):
                    sl = pl.ds(c * LANES, LANES)
                    buf_v[r, sl] = buf_v[r, sl] + pos_v[r, sl]
                return 0

            lax.fori_loop(0, S, add_row, 0)
            pltpu.sync_copy(buf_v, out_hbm.at[pl.ds(row0, rows)])
            return 0

        lax.fori_loop(0, n_chunk, chunk_body, 0)

    return pl.kernel(
        body,
        out_type=jax.ShapeDtypeStruct((N, H), jnp.float32),
        mesh=mesh,
        scratch_types=[
            pltpu.VMEM((rows,), jnp.int32),
            pltpu.VMEM((S, H), jnp.float32),
            pltpu.VMEM((rows, H), jnp.float32),
            pltpu.SemaphoreType.DMA,
        ],
        compiler_params=pltpu.CompilerParams(use_tc_tiling_on_sc=False),
    )


def kernel(x, emb_table, pos_table):
    B, S = x.shape
    V, H = emb_table.shape
    x_flat = x.reshape(B * S).astype(jnp.int32)
    out = _build(B, S, H, V)(x_flat, emb_table, pos_table[:S])
    return out.reshape(B, S, H)


# speedup vs baseline: 4.2263x; 1.3596x over previous
import functools

import jax
import jax.numpy as jnp
from jax import lax
from jax.experimental import pallas as pl
from jax.experimental.pallas import tpu as pltpu
from jax.experimental.pallas import tpu_sc as plsc

NUM_CORES = 2
NUM_SUBCORES = 16
NW = NUM_CORES * NUM_SUBCORES
LANES = 16
NG = 4
NO = 2


@functools.cache
def _build(B, S, H, V):
    N = B * S
    assert N % (NW * S) == 0, "each worker must own whole sequences"
    per_w = N // NW
    n_chunk = per_w // S
    assert n_chunk % NG == 0 and n_chunk >= NG + NO
    rows = S
    hc = H // LANES

    mesh = plsc.VectorSubcoreMesh(
        core_axis_name="c", subcore_axis_name="s",
        num_cores=NUM_CORES, num_subcores=NUM_SUBCORES)

    def body(x_hbm, emb_hbm, pos_hbm, out_hbm, idx_v, pos_v, *bufs_and_sems):
        bg = bufs_and_sems[:NG]
        bo = bufs_and_sems[NG:NG + NO]
        gsem = bufs_and_sems[NG + NO:NG + NO + NG]
        osem = bufs_and_sems[NG + NO + NG:]

        wid = lax.axis_index("s") * NUM_CORES + lax.axis_index("c")
        base = wid * per_w
        pltpu.sync_copy(pos_hbm, pos_v)
        pltpu.sync_copy(x_hbm.at[pl.ds(base, per_w)], idx_v)

        def gather(k, b):
            pltpu.async_copy(
                emb_hbm.at[idx_v.at[pl.ds(k * rows, rows)]], bg[b], gsem[b])

        for b in range(NG):
            gather(b, b)

        def outer(i, _):
            for b in range(NG):
                k = i * NG + b
                o = b % NO
                pltpu.make_async_copy(
                    emb_hbm.at[idx_v.at[pl.ds(0, rows)]], bg[b], gsem[b]).wait()

                @pl.when(k >= NO)
                def _():
                    pltpu.make_async_copy(
                        bo[o], out_hbm.at[pl.ds(0, rows)], osem[o]).wait()

                def add_row(r, _):
                    for c in range(hc):
                        sl = pl.ds(c * LANES, LANES)
                        bo[o][r, sl] = bg[b][r, sl] + pos_v[r, sl]
                    return 0

                lax.fori_loop(0, S, add_row, 0)
                pltpu.async_copy(
                    bo[o], out_hbm.at[pl.ds(base + k * rows, rows)], osem[o])

                @pl.when(k + NG < n_chunk)
                def _():
                    gather(k + NG, b)

            return 0

        lax.fori_loop(0, n_chunk // NG, outer, 0)
        for o in range(NO):
            pltpu.make_async_copy(
                bo[o], out_hbm.at[pl.ds(0, rows)], osem[o]).wait()

    buf_f32 = pltpu.VMEM((rows, H), jnp.float32)
    return pl.kernel(
        body,
        out_type=jax.ShapeDtypeStruct((N, H), jnp.float32),
        mesh=mesh,
        scratch_types=(
            [pltpu.VMEM((per_w,), jnp.int32),
             pltpu.VMEM((S, H), jnp.float32)]
            + [buf_f32] * NG
            + [buf_f32] * NO
            + [pltpu.SemaphoreType.DMA] * (NG + NO)
        ),
        compiler_params=pltpu.CompilerParams(use_tc_tiling_on_sc=False),
    )


def kernel(x, emb_table, pos_table):
    B, S = x.shape
    V, H = emb_table.shape
    x_flat = x.reshape(B * S).astype(jnp.int32)
    out = _build(B, S, H, V)(x_flat, emb_table, pos_table[:S])
    return out.reshape(B, S, H)


# speedup vs baseline: 4.2317x; 1.0013x over previous
import functools

import jax
import jax.numpy as jnp
from jax import lax
from jax.experimental import pallas as pl
from jax.experimental.pallas import tpu as pltpu
from jax.experimental.pallas import tpu_sc as plsc

NUM_CORES = 2
NUM_SUBCORES = 16
NW = NUM_CORES * NUM_SUBCORES
LANES = 16
NG = 4
NO = 2


@functools.cache
def _build(B, S, H, V):
    N = B * S
    assert N % (NW * S) == 0, "each worker must own whole sequences"
    per_w = N // NW
    n_chunk = per_w // S
    assert n_chunk % NG == 0 and n_chunk >= NG + NO
    rows = S
    hc = H // LANES

    mesh = plsc.VectorSubcoreMesh(
        core_axis_name="c", subcore_axis_name="s",
        num_cores=NUM_CORES, num_subcores=NUM_SUBCORES)

    def body(x_hbm, emb_hbm, pos_hbm, out_hbm, idx_v, pos_v, *bufs_and_sems):
        bg = bufs_and_sems[:NG]
        bo = bufs_and_sems[NG:NG + NO]
        gsem = bufs_and_sems[NG + NO:NG + NO + NG]
        osem = bufs_and_sems[NG + NO + NG:]

        wid = lax.axis_index("s") * NUM_CORES + lax.axis_index("c")
        base = wid * per_w
        pltpu.sync_copy(pos_hbm, pos_v)
        pltpu.sync_copy(x_hbm.at[pl.ds(base, per_w)], idx_v)

        def gather(k, b):
            pltpu.async_copy(
                emb_hbm.at[idx_v.at[pl.ds(k * rows, rows)]], bg[b], gsem[b])

        for b in range(NG):
            gather(b, b)

        def outer(i, _):
            for b in range(NG):
                k = i * NG + b
                o = b % NO
                pltpu.make_async_copy(
                    emb_hbm.at[idx_v.at[pl.ds(0, rows)]], bg[b], gsem[b]).wait()

                @pl.when(k >= NO)
                def _():
                    pltpu.make_async_copy(
                        bo[o], out_hbm.at[0], osem[o]).wait()

                def add_row(r, _):
                    for c in range(hc):
                        sl = pl.ds(c * LANES, LANES)
                        bo[o][r, sl] = bg[b][r, sl] + pos_v[r, sl]
                    return 0

                lax.fori_loop(0, S, add_row, 0)
                pltpu.async_copy(
                    bo[o], out_hbm.at[wid * n_chunk + k], osem[o])

                @pl.when(k + NG < n_chunk)
                def _():
                    gather(k + NG, b)

            return 0

        lax.fori_loop(0, n_chunk // NG, outer, 0)
        for o in range(NO):
            pltpu.make_async_copy(bo[o], out_hbm.at[0], osem[o]).wait()

    buf_f32 = pltpu.VMEM((rows, H), jnp.float32)
    return pl.kernel(
        body,
        out_type=jax.ShapeDtypeStruct((B, S, H), jnp.float32),
        mesh=mesh,
        scratch_types=(
            [pltpu.VMEM((per_w,), jnp.int32),
             pltpu.VMEM((S, H), jnp.float32)]
            + [buf_f32] * NG
            + [buf_f32] * NO
            + [pltpu.SemaphoreType.DMA] * (NG + NO)
        ),
        compiler_params=pltpu.CompilerParams(use_tc_tiling_on_sc=False),
    )


def kernel(x, emb_table, pos_table):
    B, S = x.shape
    V, H = emb_table.shape
    x_flat = x.reshape(B * S).astype(jnp.int32)
    return _build(B, S, H, V)(x_flat, emb_table, pos_table[:S])


# speedup vs baseline: 4.6635x; 1.1020x over previous
import functools

import jax
import jax.numpy as jnp
from jax import lax
from jax.experimental import pallas as pl
from jax.experimental.pallas import tpu as pltpu
from jax.experimental.pallas import tpu_sc as plsc

NUM_CORES = 2
NUM_SUBCORES = 16
NW = NUM_CORES * NUM_SUBCORES
LANES = 16
HP = 128
ROWS = 40


@functools.cache
def _build(B, S, H, V):
    N = B * S
    nb = S // ROWS
    seq_w = B // NW
    n_chunk = seq_w * nb
    assert B % NW == 0 and S % ROWS == 0 and ROWS % 8 == 0
    hc = H // LANES

    mesh = plsc.VectorSubcoreMesh(
        core_axis_name="c", subcore_axis_name="s",
        num_cores=NUM_CORES, num_subcores=NUM_SUBCORES)

    def body(x_hbm, emb_hbm, pos_hbm, out_hbm, idx_v, pos_v, *rest):
        bg = rest[:nb]
        bo = rest[nb:2 * nb]
        gsem = rest[2 * nb:3 * nb]
        osem = rest[3 * nb:]

        wid = lax.axis_index("s") * NUM_CORES + lax.axis_index("c")
        per_w = seq_w * S
        pltpu.sync_copy(pos_hbm, pos_v)
        pltpu.sync_copy(x_hbm.at[pl.ds(wid * per_w, per_w)], idx_v)

        def gather(q, b):
            pltpu.async_copy(
                emb_hbm.at[idx_v.at[pl.ds(q * S + b * ROWS, ROWS)]], bg[b],
                gsem[b])

        for b in range(nb):
            gather(0, b)

        def outer(i, _):
            for b in range(nb):
                j = i * nb + b
                pltpu.make_async_copy(
                    emb_hbm.at[idx_v.at[pl.ds(b * ROWS, ROWS)]], bg[b],
                    gsem[b]).wait()

                @pl.when(j >= nb)
                def _():
                    pltpu.make_async_copy(
                        bo[b], out_hbm.at[0, pl.ds(b * ROWS, ROWS)],
                        osem[b]).wait()

                def add_row(r, _):
                    for c in range(hc):
                        sl = pl.ds(c * LANES, LANES)
                        bo[b][r, sl] = bg[b][r, sl] + pos_v[b * ROWS + r, sl]
                    return 0

                lax.fori_loop(0, ROWS, add_row, 0)
                pltpu.async_copy(
                    bo[b], out_hbm.at[wid * seq_w + i,
                                      pl.ds(b * ROWS, ROWS)], osem[b])

                @pl.when(i + 1 < seq_w)
                def _():
                    gather(i + 1, b)

            return 0

        lax.fori_loop(0, seq_w, outer, 0)
        for b in range(nb):
            pltpu.make_async_copy(
                bo[b], out_hbm.at[0, pl.ds(b * ROWS, ROWS)], osem[b]).wait()

    return pl.kernel(
        body,
        out_type=jax.ShapeDtypeStruct((B, S, H), jnp.float32),
        mesh=mesh,
        scratch_types=(
            [pltpu.VMEM((seq_w * S,), jnp.int32),
             pltpu.VMEM((S, H), jnp.float32)]
            + [pltpu.VMEM((ROWS, HP), jnp.float32)] * nb
            + [pltpu.VMEM((ROWS, H), jnp.float32)] * nb
            + [pltpu.SemaphoreType.DMA] * (2 * nb)
        ),
    )


def kernel(x, emb_table, pos_table):
    B, S = x.shape
    V, H = emb_table.shape
    embp = jnp.pad(emb_table, ((0, 0), (0, HP - H)))
    x_flat = x.reshape(B * S).astype(jnp.int32)
    return _build(B, S, H, V)(x_flat, embp, pos_table[:S])
